# baseline (speedup 1.0000x reference)
import functools

import jax
import jax.numpy as jnp
from jax import lax
from jax.experimental import pallas as pl
from jax.experimental.pallas import tpu as pltpu
from jax.experimental.pallas import tpu_sc as plsc

B, H, S, D, T = 8, 16, 2048, 128, 16
BHS = B * H * S

NC, NS = 2, 16
NW = NC * NS
ROWS = B * H * T
RPW = ROWS // NW
GPW = RPW // T

CPW = BHS // NW
CH = 256
NBUF = 2
NCHUNK = CPW // CH

_mesh = plsc.VectorSubcoreMesh(core_axis_name="c", subcore_axis_name="s")


@functools.partial(
    pl.kernel,
    out_type=(jax.ShapeDtypeStruct((BHS, D), jnp.float32),
              jax.ShapeDtypeStruct((BHS, D), jnp.float32)),
    mesh=_mesh,
    scratch_types=[
        pltpu.VMEM((NBUF, CH, D), jnp.float32),
        pltpu.VMEM((T,), jnp.int32),
        pltpu.VMEM((RPW,), jnp.int32),
        pltpu.VMEM((RPW, D), jnp.float32),
        pltpu.VMEM((RPW, D), jnp.float32),
        pltpu.SemaphoreType.DMA((NBUF,)),
        pltpu.SemaphoreType.DMA((NBUF,)),
        pltpu.SemaphoreType.DMA,
        pltpu.SemaphoreType.DMA,
    ],
)
def _cache_update(ck_hbm, cv_hbm, nk_hbm, nv_hbm, pos_hbm, ok_hbm, ov_hbm,
                  ring, pos_v, idx_v, krows_v, vrows_v,
                  insems, outsems, semk, semv):
    wid = lax.axis_index("s") * NC + lax.axis_index("c")
    base = wid * CPW
    rbase = wid * RPW

    cpk_in = pltpu.async_copy(nk_hbm.at[pl.ds(rbase, RPW)], krows_v, semk)
    cpv_in = pltpu.async_copy(nv_hbm.at[pl.ds(rbase, RPW)], vrows_v, semv)
    pltpu.sync_copy(pos_hbm, pos_v)
    pos = pos_v[...]
    for g in range(GPW):
        bh = wid * GPW + g
        idx_v[pl.ds(g * T, T)] = pos + bh * S

    def copy_slice(src, dst, first):
        @pl.loop(0, NCHUNK, step=NBUF)
        def _(i0):
            for b in range(NBUF):
                off = base + (i0 + b) * CH
                @pl.when(jnp.logical_or(i0 > 0, jnp.logical_not(first)))
                def _():
                    pltpu.make_async_copy(
                        ring.at[b], dst.at[pl.ds(base, CH)], outsems.at[b]
                    ).wait()
                pltpu.async_copy(src.at[pl.ds(off, CH)], ring.at[b],
                                 insems.at[b])
            for b in range(NBUF):
                off = base + (i0 + b) * CH
                pltpu.make_async_copy(src.at[pl.ds(off, CH)], ring.at[b],
                                      insems.at[b]).wait()
                pltpu.async_copy(ring.at[b], dst.at[pl.ds(off, CH)],
                                 outsems.at[b])

    copy_slice(ck_hbm, ok_hbm, first=jnp.bool_(True))
    copy_slice(cv_hbm, ov_hbm, first=jnp.bool_(False))
    for b in range(NBUF):
        pltpu.make_async_copy(ring.at[b], ov_hbm.at[pl.ds(base, CH)],
                              outsems.at[b]).wait()

    cpk_in.wait()
    cpv_in.wait()
    cpk = pltpu.async_copy(krows_v, ok_hbm.at[idx_v], semk)
    cpv = pltpu.async_copy(vrows_v, ov_hbm.at[idx_v], semv)
    cpk.wait()
    cpv.wait()


def kernel(cache_k, cache_v, new_k, new_v, position_ids):
    ok, ov = _cache_update(cache_k.reshape(BHS, D),
                           cache_v.reshape(BHS, D),
                           new_k.reshape(ROWS, D),
                           new_v.reshape(ROWS, D),
                           position_ids.astype(jnp.int32))
    return (ok.reshape(B, H, S, D), ov.reshape(B, H, S, D))


# speedup vs baseline: 1.1562x; 1.1562x over previous
import functools

import jax
import jax.numpy as jnp
from jax import lax
from jax.experimental import pallas as pl
from jax.experimental.pallas import tpu as pltpu
from jax.experimental.pallas import tpu_sc as plsc

B, H, S, D, T = 8, 16, 2048, 128, 16
BHS = B * H * S

NC, NS = 1, 16
NW = NC * NS
ROWS = B * H * T
RPW = ROWS // NW
GPW = RPW // T

_mesh = plsc.VectorSubcoreMesh(core_axis_name="c", subcore_axis_name="s", num_cores=1)


@functools.partial(
    pl.kernel,
    out_type=(),
    mesh=_mesh,
    scratch_types=[
        pltpu.VMEM((T,), jnp.int32),
        pltpu.VMEM((RPW,), jnp.int32),
        pltpu.VMEM((RPW, D), jnp.float32),
        pltpu.VMEM((RPW, D), jnp.float32),
        pltpu.SemaphoreType.DMA,
        pltpu.SemaphoreType.DMA,
    ],
)
def _scatter_update(ck_ref, cv_ref, nk_hbm, nv_hbm, pos_hbm,
                    pos_v, idx_v, krows_v, vrows_v, semk, semv):
    wid = lax.axis_index("s") * NC + lax.axis_index("c")
    base = wid * RPW
    cpk_in = pltpu.async_copy(nk_hbm.at[pl.ds(base, RPW)], krows_v, semk)
    cpv_in = pltpu.async_copy(nv_hbm.at[pl.ds(base, RPW)], vrows_v, semv)
    pltpu.sync_copy(pos_hbm, pos_v)
    pos = pos_v[...]
    for g in range(GPW):
        bh = wid * GPW + g
        idx_v[pl.ds(g * T, T)] = pos + bh * S
    cpk_in.wait()
    cpv_in.wait()
    cpk = pltpu.async_copy(krows_v, ck_ref.at[idx_v], semk)
    cpv = pltpu.async_copy(vrows_v, cv_ref.at[idx_v], semv)
    cpk.wait()
    cpv.wait()


def kernel(cache_k, cache_v, new_k, new_v, position_ids):
    pos = position_ids.astype(jnp.int32)
    ck = jax.new_ref(cache_k.reshape(BHS, D))
    cv = jax.new_ref(cache_v.reshape(BHS, D))
    _scatter_update(ck, cv,
                    new_k.reshape(ROWS, D),
                    new_v.reshape(ROWS, D),
                    pos)
    return (ck[...].reshape(B, H, S, D), cv[...].reshape(B, H, S, D))
